# baseline (speedup 1.0000x reference)
import functools

import jax
import jax.numpy as jnp
from jax import lax
from jax.experimental import pallas as pl
from jax.experimental.pallas import tpu as pltpu
from jax.experimental.pallas import tpu_sc as plsc

GROUP_SIZE = 32
BQ = 256

NC = 2
NS = 16
NW = NC * NS

BIGI = 1 << 30
FINF = float("inf")


def _dist_kernel(xyzq_ref, xyz_ref, dist_ref, tau_ref, *, n):
    q = xyzq_ref[0]
    ka = xyz_ref[0]
    sqq = jnp.sum(q * q, axis=1)
    sqk = jnp.sum(ka * ka, axis=1)
    inner = lax.dot_general(q, ka, (((1,), (1,)), ((), ())),
                            preferred_element_type=jnp.float32)
    dist = (sqq[:, None] + sqk[None, :]) - 2.0 * inner
    dist_ref[0] = dist
    nch = n // 128
    tau = jnp.min(lax.slice_in_dim(dist, 0, 128, axis=1), axis=1)
    for ci in range(1, nch):
        m = jnp.min(lax.slice_in_dim(dist, ci * 128, (ci + 1) * 128, axis=1),
                    axis=1)
        tau = jnp.maximum(tau, m)
    tau_ref[0, 0, :] = tau


def _dist_tc(xyz):
    b, n, _ = xyz.shape
    grid = (b, n // BQ)
    nq = n // BQ
    return pl.pallas_call(
        functools.partial(_dist_kernel, n=n),
        grid=grid,
        in_specs=[
            pl.BlockSpec((1, BQ, 3), lambda bi, qi: (bi, qi, 0)),
            pl.BlockSpec((1, n, 3), lambda bi, qi: (bi, 0, 0)),
        ],
        out_specs=(
            pl.BlockSpec((1, BQ, n), lambda bi, qi: (bi, qi, 0)),
            pl.BlockSpec((1, 1, BQ), lambda bi, qi: (bi * nq + qi, 0, 0)),
        ),
        out_shape=(
            jax.ShapeDtypeStruct((b, n, n), jnp.float32),
            jax.ShapeDtypeStruct((b * nq, 1, BQ), jnp.float32),
        ),
    )(xyz, xyz)


def _sc_select_gather(dist2, tau1, feat_flat, xyz2, *, b, n, k, c):
    rtot = b * n * k
    rw = rtot // NW
    pw = rw // k
    nvreg = n // 16
    cap = n + 32

    mesh = plsc.VectorSubcoreMesh(core_axis_name="c", subcore_axis_name="s",
                                  num_cores=NC, num_subcores=NS)

    @functools.partial(
        pl.kernel, mesh=mesh,
        compiler_params=pltpu.CompilerParams(needs_layout_passes=False),
        out_type=(
            jax.ShapeDtypeStruct((rtot, c), jnp.float32),
            jax.ShapeDtypeStruct((rtot * 3,), jnp.float32),
        ),
        scratch_types=[
            pltpu.VMEM((n,), jnp.float32),
            pltpu.VMEM((n,), jnp.float32),
            pltpu.VMEM((cap,), jnp.int32),
            pltpu.VMEM((cap,), jnp.float32),
            pltpu.VMEM((256,), jnp.float32),
            pltpu.VMEM((pw + 16,), jnp.float32),
            pltpu.VMEM((n * 3,), jnp.float32),
            pltpu.VMEM((rw * 3,), jnp.float32),
            pltpu.VMEM((k,), jnp.int32),
            pltpu.VMEM((k,), jnp.int32),
            pltpu.VMEM((k, c), jnp.float32),
            pltpu.VMEM((k, c), jnp.float32),
            pltpu.SemaphoreType.DMA,
            pltpu.SemaphoreType.DMA,
            pltpu.SemaphoreType.DMA,
            pltpu.SemaphoreType.DMA,
        ],
    )
    def body(dist_hbm, tau_hbm, feat_hbm, xyz_hbm, feat_out, neigh_out,
             dr0, dr1, ci_v, cv_v, m1_v, tau_v, xyz_v, nst_v,
             ir0, ir1, fb0, fb1, ds0, ds1, fs0, fs1):
        wid = lax.axis_index("s") * NC + lax.axis_index("c")
        bi = wid // (n // pw)
        row0 = wid * pw
        base = wid * rw
        boff = bi * n

        iota16 = lax.broadcasted_iota(jnp.int32, (16,), 0)
        zer16 = jnp.zeros((16,), jnp.int32)
        zer16f = jnp.zeros((16,), jnp.float32)
        inf16 = jnp.full((16,), FINF, jnp.float32)

        drs = (dr0, dr1)
        dsems = (ds0, ds1)
        irs = (ir0, ir1)
        fbs = (fb0, fb1)
        fsems = (fs0, fs1)

        pltpu.sync_copy(tau_hbm.at[pl.ds(row0, pw)], tau_v.at[pl.ds(0, pw)])
        pltpu.sync_copy(xyz_hbm.at[bi], xyz_v)
        pltpu.async_copy(dist_hbm.at[row0], dr0, ds0)
        pltpu.async_copy(dist_hbm.at[row0 + 1], dr1, ds1)

        def process_point(r, slot):
            dr = drs[slot]
            pltpu.make_async_copy(dist_hbm.at[row0 + r], dr, dsems[slot]).wait()

            tv = tau_v[pl.ds(r, 16)]
            tau_s = jnp.min(jnp.where(iota16 == 0, tv, inf16))
            tau16 = zer16f + tau_s

            def comp_body(i, carry):
                cnt, civ = carry
                v = dr[pl.ds(i * 16, 16)]
                msk = v <= tau16
                plsc.store_compressed(ci_v.at[pl.ds(cnt, 16)], civ, mask=msk)
                cnt = cnt + jnp.max(plsc.all_reduce_population_count(msk))
                return cnt, civ + 16

            cnt, _ = lax.fori_loop(0, nvreg, comp_body, (jnp.int32(0), iota16))
            ci_v[pl.ds(cnt, 16)] = zer16

            for g in range(16):
                m1_v[pl.ds(g * 16, 16)] = inf16

            nv = (cnt + 15) // 16
            cnt16 = zer16 + cnt

            def build_body(g, _):
                civ = ci_v[pl.ds(g * 16, 16)]
                cvv = plsc.load_gather(dr, [civ])
                valid = (g * 16 + iota16) < cnt16
                cvv = jnp.where(valid, cvv, inf16)
                cv_v[pl.ds(g * 16, 16)] = cvv
                m = jnp.min(cvv)
                g2 = g // 16
                l1 = g - g2 * 16
                m1v = m1_v[pl.ds(g2 * 16, 16)]
                m1_v[pl.ds(g2 * 16, 16)] = jnp.where(iota16 == zer16 + l1,
                                                     zer16f + m, m1v)
                return 0

            lax.fori_loop(0, nv, build_body, 0)

            m2 = inf16
            for g2 in range(16):
                mm = jnp.min(m1_v[pl.ds(g2 * 16, 16)])
                m2 = jnp.where(iota16 == g2, zer16f + mm, m2)

            acc0 = zer16
            acc1 = zer16
            for j in range(k):
                gm = jnp.min(m2)
                gm16 = zer16f + gm
                w2 = jnp.max(plsc.all_reduce_ffs(m2 == gm16))
                m1v = m1_v[pl.ds(w2 * 16, 16)]
                w1 = jnp.max(plsc.all_reduce_ffs(m1v == gm16))
                w = w2 * 16 + w1
                cvv = cv_v[pl.ds(w * 16, 16)]
                civ = ci_v[pl.ds(w * 16, 16)]
                selidx = jnp.min(jnp.where(cvv == gm16, civ, zer16 + BIGI))
                sel16 = zer16 + selidx
                cvv2 = jnp.where(civ == sel16, inf16, cvv)
                cv_v[pl.ds(w * 16, 16)] = cvv2
                nm1 = jnp.min(cvv2)
                m1v2 = jnp.where(iota16 == zer16 + w1, zer16f + nm1, m1v)
                m1_v[pl.ds(w2 * 16, 16)] = m1v2
                nm2 = jnp.min(m1v2)
                m2 = jnp.where(iota16 == zer16 + w2, zer16f + nm2, m2)
                if j < 16:
                    acc0 = jnp.where(iota16 == j, sel16, acc0)
                else:
                    acc1 = jnp.where(iota16 == j - 16, sel16, acc1)

            cp = (row0 - bi * n + r) * 3
            for h, acc in ((0, acc0), (1, acc1)):
                rl = (r * k + 16 * h) * 3 + iota16 * 3
                for d in range(3):
                    xs = plsc.load_gather(xyz_v, [acc * 3 + d])
                    cs = plsc.load_gather(xyz_v, [zer16 + (cp + d)])
                    plsc.store_scatter(nst_v, [rl + d], xs - cs)
                irs[slot][pl.ds(16 * h, 16)] = acc + boff

            pltpu.async_copy(feat_hbm.at[irs[slot]], fbs[slot], fsems[slot])

            @pl.when(r >= 1)
            def _drain():
                other = 1 - slot
                pltpu.make_async_copy(feat_hbm.at[irs[other]], fbs[other],
                                      fsems[other]).wait()
                pltpu.sync_copy(fbs[other],
                                feat_out.at[pl.ds(base + (r - 1) * k, k)])

            @pl.when(r + 2 < pw)
            def _prefetch():
                pltpu.async_copy(dist_hbm.at[row0 + r + 2], dr, dsems[slot])

        @pl.loop(0, pw, step=2)
        def _(r0):
            for slot in range(2):
                process_point(r0 + slot, slot)

        lslot = (pw - 1) % 2
        pltpu.make_async_copy(feat_hbm.at[irs[lslot]], fbs[lslot],
                              fsems[lslot]).wait()
        pltpu.sync_copy(fbs[lslot],
                        feat_out.at[pl.ds(base + (pw - 1) * k, k)])
        pltpu.sync_copy(nst_v, neigh_out.at[pl.ds(base * 3, rw * 3)])

    return body(dist2, tau1, feat_flat, xyz2)


def kernel(xyz, feat):
    b, n, _ = xyz.shape
    c = feat.shape[-1]
    k = GROUP_SIZE
    dist, tau = _dist_tc(xyz)
    nfeat, neigh = _sc_select_gather(
        dist.reshape(b * n, n), tau.reshape(b * n), feat.reshape(b * n, c),
        xyz.reshape(b, n * 3), b=b, n=n, k=k, c=c)
    return (neigh.reshape(b, n, k, 3), nfeat.reshape(b, n, k, c))


# speedup vs baseline: 1.8442x; 1.8442x over previous
import functools

import jax
import jax.numpy as jnp
from jax import lax
from jax.experimental import pallas as pl
from jax.experimental.pallas import tpu as pltpu
from jax.experimental.pallas import tpu_sc as plsc

GROUP_SIZE = 32
BQ = 256

NC = 2
NS = 16
NW = NC * NS

BIGI = 1 << 30
FINF = float("inf")


def _dist_kernel(xyzq_ref, xyz_ref, dist_ref, tau_ref, *, n):
    q = xyzq_ref[0]
    ka = xyz_ref[0]
    sqq = jnp.sum(q * q, axis=1)
    sqk = jnp.sum(ka * ka, axis=1)
    inner = lax.dot_general(q, ka, (((1,), (1,)), ((), ())),
                            preferred_element_type=jnp.float32)
    dist = (sqq[:, None] + sqk[None, :]) - 2.0 * inner
    dist_ref[...] = dist
    nch = n // 128
    tau = jnp.min(lax.slice_in_dim(dist, 0, 128, axis=1), axis=1)
    for ci in range(1, nch):
        m = jnp.min(lax.slice_in_dim(dist, ci * 128, (ci + 1) * 128, axis=1),
                    axis=1)
        tau = jnp.maximum(tau, m)
    tau_ref[0, 0, :] = tau


def _dist_tc(xyz):
    b, n, _ = xyz.shape
    grid = (b, n // BQ)
    nq = n // BQ
    return pl.pallas_call(
        functools.partial(_dist_kernel, n=n),
        grid=grid,
        in_specs=[
            pl.BlockSpec((1, BQ, 3), lambda bi, qi: (bi, qi, 0)),
            pl.BlockSpec((1, n, 3), lambda bi, qi: (bi, 0, 0)),
        ],
        out_specs=(
            pl.BlockSpec((BQ, n), lambda bi, qi: (bi * nq + qi, 0)),
            pl.BlockSpec((1, 1, BQ), lambda bi, qi: (bi * nq + qi, 0, 0)),
        ),
        out_shape=(
            jax.ShapeDtypeStruct((b * n, n), jnp.float32),
            jax.ShapeDtypeStruct((b * nq, 1, BQ), jnp.float32),
        ),
    )(xyz, xyz)


def _sc_select_gather(dist2, tau1, feat_flat, xyz2, *, b, n, k, c):
    rtot = b * n * k
    rw = rtot // NW
    pw = rw // k
    nvreg = n // 16
    cap = n + 32

    mesh = plsc.VectorSubcoreMesh(core_axis_name="c", subcore_axis_name="s",
                                  num_cores=NC, num_subcores=NS)

    @functools.partial(
        pl.kernel, mesh=mesh,
        compiler_params=pltpu.CompilerParams(needs_layout_passes=False),
        out_type=(
            jax.ShapeDtypeStruct((rtot, c), jnp.float32),
            jax.ShapeDtypeStruct((rtot * 3,), jnp.float32),
        ),
        scratch_types=[
            pltpu.VMEM((n,), jnp.float32),
            pltpu.VMEM((n,), jnp.float32),
            pltpu.VMEM((cap,), jnp.int32),
            pltpu.VMEM((pw + 16,), jnp.float32),
            pltpu.VMEM((n * 3,), jnp.float32),
            pltpu.VMEM((rw * 3,), jnp.float32),
            pltpu.VMEM((k,), jnp.int32),
            pltpu.VMEM((k,), jnp.int32),
            pltpu.VMEM((k, c), jnp.float32),
            pltpu.VMEM((k, c), jnp.float32),
            pltpu.SemaphoreType.DMA,
            pltpu.SemaphoreType.DMA,
            pltpu.SemaphoreType.DMA,
            pltpu.SemaphoreType.DMA,
        ],
    )
    def body(dist_hbm, tau_hbm, feat_hbm, xyz_hbm, feat_out, neigh_out,
             dr0, dr1, ci_v, tau_v, xyz_v, nst_v,
             ir0, ir1, fb0, fb1, ds0, ds1, fs0, fs1):
        wid = lax.axis_index("s") * NC + lax.axis_index("c")
        bi = wid // (n // pw)
        row0 = wid * pw
        base = wid * rw
        boff = bi * n

        iota16 = lax.broadcasted_iota(jnp.int32, (16,), 0)
        zer16 = jnp.zeros((16,), jnp.int32)
        zer16f = jnp.zeros((16,), jnp.float32)
        inf16 = jnp.full((16,), FINF, jnp.float32)

        drs = (dr0, dr1)
        dsems = (ds0, ds1)
        irs = (ir0, ir1)
        fbs = (fb0, fb1)
        fsems = (fs0, fs1)

        pltpu.sync_copy(tau_hbm.at[pl.ds(row0, pw)], tau_v.at[pl.ds(0, pw)])
        pltpu.sync_copy(xyz_hbm.at[bi], xyz_v)
        pltpu.async_copy(dist_hbm.at[row0], dr0, ds0)
        pltpu.async_copy(dist_hbm.at[row0 + 1], dr1, ds1)

        def process_point(r, slot):
            dr = drs[slot]
            pltpu.make_async_copy(dist_hbm.at[row0 + r], dr, dsems[slot]).wait()

            tv = tau_v[pl.ds(r, 16)]
            tau16 = zer16f + tv[0]

            def comp_body(i, carry):
                cnt, civ = carry
                v = dr[pl.ds(i * 16, 16)]
                msk = v <= tau16
                plsc.store_compressed(ci_v.at[pl.ds(cnt, 16)], civ, mask=msk)
                pc = plsc.all_reduce_population_count(msk)
                return cnt + pc[0], civ + 16

            cnt, _ = lax.fori_loop(0, nvreg, comp_body, (jnp.int32(0), iota16),
                                   unroll=8)
            ci_v[pl.ds(cnt, 16)] = zer16
            nv = (cnt + 15) // 16
            cnt16 = zer16 + cnt

            def leaf(g, carry):
                tk0, tk1, tv0_, tv1_ = carry
                civ = ci_v[pl.ds(g * 16, 16)]
                kv = plsc.load_gather(dr, [civ])
                valid = (g * 16 + iota16) < cnt16
                kv = jnp.where(valid, kv, inf16)
                ks, vs = plsc.sort_key_val(kv, civ)
                rk = jnp.flip(ks)
                rv = jnp.flip(vs)
                m = tk1 <= rk
                ck = jnp.where(m, tk1, rk)
                cw = jnp.where(m, tv1_, rv)
                ck, cw = plsc.sort_key_val(ck, cw)
                rk2 = jnp.flip(ck)
                rv2 = jnp.flip(cw)
                m2 = tk0 <= rk2
                lok = jnp.where(m2, tk0, rk2)
                lov = jnp.where(m2, tv0_, rv2)
                hik = jnp.where(m2, rk2, tk0)
                hiv = jnp.where(m2, rv2, tv0_)
                tk0, tv0_ = plsc.sort_key_val(lok, lov)
                tk1, tv1_ = plsc.sort_key_val(hik, hiv)
                return tk0, tk1, tv0_, tv1_

            _, _, acc0, acc1 = lax.fori_loop(
                0, nv, leaf, (inf16, inf16, zer16, zer16))

            cp = (row0 - bi * n + r) * 3
            for h, acc in ((0, acc0), (1, acc1)):
                rl = (r * k + 16 * h) * 3 + iota16 * 3
                for d in range(3):
                    xs = plsc.load_gather(xyz_v, [acc * 3 + d])
                    cs = plsc.load_gather(xyz_v, [zer16 + (cp + d)])
                    plsc.store_scatter(nst_v, [rl + d], xs - cs)
                irs[slot][pl.ds(16 * h, 16)] = acc + boff

            pltpu.async_copy(feat_hbm.at[irs[slot]], fbs[slot], fsems[slot])

            @pl.when(r >= 1)
            def _drain():
                other = 1 - slot
                pltpu.make_async_copy(feat_hbm.at[irs[other]], fbs[other],
                                      fsems[other]).wait()
                pltpu.sync_copy(fbs[other],
                                feat_out.at[pl.ds(base + (r - 1) * k, k)])

            @pl.when(r + 2 < pw)
            def _prefetch():
                pltpu.async_copy(dist_hbm.at[row0 + r + 2], dr, dsems[slot])

        @pl.loop(0, pw, step=2)
        def _(r0):
            for slot in range(2):
                process_point(r0 + slot, slot)

        lslot = (pw - 1) % 2
        pltpu.make_async_copy(feat_hbm.at[irs[lslot]], fbs[lslot],
                              fsems[lslot]).wait()
        pltpu.sync_copy(fbs[lslot],
                        feat_out.at[pl.ds(base + (pw - 1) * k, k)])
        pltpu.sync_copy(nst_v, neigh_out.at[pl.ds(base * 3, rw * 3)])

    return body(dist2, tau1, feat_flat, xyz2)


def kernel(xyz, feat):
    b, n, _ = xyz.shape
    c = feat.shape[-1]
    k = GROUP_SIZE
    dist, tau = _dist_tc(xyz)
    nfeat, neigh = _sc_select_gather(
        dist.reshape(b * n, n), tau.reshape(b * n), feat.reshape(b * n, c),
        xyz.reshape(b, n * 3), b=b, n=n, k=k, c=c)
    return (neigh.reshape(b, n, k, 3), nfeat.reshape(b, n, k, c))


# speedup vs baseline: 2.4740x; 1.3415x over previous
import functools

import jax
import jax.numpy as jnp
from jax import lax
from jax.experimental import pallas as pl
from jax.experimental.pallas import tpu as pltpu
from jax.experimental.pallas import tpu_sc as plsc

GROUP_SIZE = 32
BQ = 256

NC = 2
NS = 16
NW = NC * NS

BIGI = 1 << 30
FINF = float("inf")


def _dist_kernel(xyzq_ref, xyz_ref, dist_ref, tau_ref, *, n):
    q = xyzq_ref[0]
    ka = xyz_ref[0]
    sqq = jnp.sum(q * q, axis=1)
    sqk = jnp.sum(ka * ka, axis=1)
    inner = lax.dot_general(q, ka, (((1,), (1,)), ((), ())),
                            preferred_element_type=jnp.float32)
    dist = (sqq[:, None] + sqk[None, :]) - 2.0 * inner
    dist_ref[...] = dist
    nch = n // 128
    tau = jnp.min(lax.slice_in_dim(dist, 0, 128, axis=1), axis=1)
    for ci in range(1, nch):
        m = jnp.min(lax.slice_in_dim(dist, ci * 128, (ci + 1) * 128, axis=1),
                    axis=1)
        tau = jnp.maximum(tau, m)
    tau_ref[0, 0, :] = tau


def _dist_tc(xyz):
    b, n, _ = xyz.shape
    grid = (b, n // BQ)
    nq = n // BQ
    return pl.pallas_call(
        functools.partial(_dist_kernel, n=n),
        grid=grid,
        in_specs=[
            pl.BlockSpec((1, BQ, 3), lambda bi, qi: (bi, qi, 0)),
            pl.BlockSpec((1, n, 3), lambda bi, qi: (bi, 0, 0)),
        ],
        out_specs=(
            pl.BlockSpec((BQ, n), lambda bi, qi: (bi * nq + qi, 0)),
            pl.BlockSpec((1, 1, BQ), lambda bi, qi: (bi * nq + qi, 0, 0)),
        ),
        out_shape=(
            jax.ShapeDtypeStruct((b * n, n), jnp.float32),
            jax.ShapeDtypeStruct((b * nq, 1, BQ), jnp.float32),
        ),
    )(xyz, xyz)


def _sc_select_gather(dist2, tau1, feat_flat, xyz2, *, b, n, k, c):
    rtot = b * n * k
    rw = rtot // NW
    pw = rw // k
    nvreg = n // 16
    cap = n + 32

    mesh = plsc.VectorSubcoreMesh(core_axis_name="c", subcore_axis_name="s",
                                  num_cores=NC, num_subcores=NS)

    @functools.partial(
        pl.kernel, mesh=mesh,
        compiler_params=pltpu.CompilerParams(needs_layout_passes=False),
        out_type=(
            jax.ShapeDtypeStruct((rtot, c), jnp.float32),
            jax.ShapeDtypeStruct((rtot * 3,), jnp.float32),
        ),
        scratch_types=[
            pltpu.VMEM((n,), jnp.float32),
            pltpu.VMEM((n,), jnp.float32),
            pltpu.VMEM((cap,), jnp.int32),
            pltpu.VMEM((cap,), jnp.int32),
            pltpu.VMEM((pw + 16,), jnp.float32),
            pltpu.VMEM((n * 3,), jnp.float32),
            pltpu.VMEM((rw * 3,), jnp.float32),
            pltpu.VMEM((k,), jnp.int32),
            pltpu.VMEM((k,), jnp.int32),
            pltpu.VMEM((k, c), jnp.float32),
            pltpu.VMEM((k, c), jnp.float32),
            pltpu.SemaphoreType.DMA,
            pltpu.SemaphoreType.DMA,
            pltpu.SemaphoreType.DMA,
            pltpu.SemaphoreType.DMA,
            pltpu.SemaphoreType.DMA,
            pltpu.SemaphoreType.DMA,
        ],
    )
    def body(dist_hbm, tau_hbm, feat_hbm, xyz_hbm, feat_out, neigh_out,
             dr0, dr1, cia_v, cib_v, tau_v, xyz_v, nst_v,
             ir0, ir1, fb0, fb1, ds0, ds1, fs0, fs1, ws0, ws1):
        wid = lax.axis_index("s") * NC + lax.axis_index("c")
        bi = wid // (n // pw)
        row0 = wid * pw
        base = wid * rw
        boff = bi * n

        iota16 = lax.broadcasted_iota(jnp.int32, (16,), 0)
        zer16 = jnp.zeros((16,), jnp.int32)
        zer16f = jnp.zeros((16,), jnp.float32)
        inf16 = jnp.full((16,), FINF, jnp.float32)

        drs = (dr0, dr1)
        dsems = (ds0, ds1)
        irs = (ir0, ir1)
        fbs = (fb0, fb1)
        fsems = (fs0, fs1)
        wsems = (ws0, ws1)

        pltpu.sync_copy(tau_hbm.at[pl.ds(row0, pw)], tau_v.at[pl.ds(0, pw)])
        pltpu.sync_copy(xyz_hbm.at[bi], xyz_v)
        pltpu.async_copy(dist_hbm.at[row0], dr0, ds0)
        pltpu.async_copy(dist_hbm.at[row0 + 1], dr1, ds1)

        def process_point(r, slot):
            dr = drs[slot]
            pltpu.make_async_copy(dist_hbm.at[row0 + r], dr, dsems[slot]).wait()

            tv = tau_v[pl.ds(r, 16)]
            tau16 = zer16f + tv[0]

            half = n // 2

            def comp_body(i, carry):
                cna, cnb, civ = carry
                va = dr[pl.ds(i * 16, 16)]
                vb = dr[pl.ds(half + i * 16, 16)]
                mska = va <= tau16
                mskb = vb <= tau16
                plsc.store_compressed(cia_v.at[pl.ds(cna, 16)], civ,
                                      mask=mska)
                plsc.store_compressed(cib_v.at[pl.ds(cnb, 16)], civ + half,
                                      mask=mskb)
                pca = plsc.all_reduce_population_count(mska)
                pcb = plsc.all_reduce_population_count(mskb)
                return cna + pca[0], cnb + pcb[0], civ + 16

            cna, cnb, _ = lax.fori_loop(
                0, half // 16, comp_body,
                (jnp.int32(0), jnp.int32(0), iota16), unroll=4)
            cia_v[pl.ds(cna, 16)] = zer16
            cib_v[pl.ds(cnb, 16)] = zer16

            def make_leaf(buf, cnt16):
                def leaf(g, carry):
                    tk0, tk1, tv0_, tv1_ = carry
                    civ = buf[pl.ds(g * 16, 16)]
                    kv = plsc.load_gather(dr, [civ])
                    valid = (g * 16 + iota16) < cnt16
                    kv = jnp.where(valid, kv, inf16)
                    ks, vs = plsc.sort_key_val(kv, civ)
                    rk = jnp.flip(ks)
                    rv = jnp.flip(vs)
                    m = tk1 <= rk
                    ck = jnp.where(m, tk1, rk)
                    cw = jnp.where(m, tv1_, rv)
                    ck, cw = plsc.sort_key_val(ck, cw)
                    rk2 = jnp.flip(ck)
                    rv2 = jnp.flip(cw)
                    m2 = tk0 <= rk2
                    lok = jnp.where(m2, tk0, rk2)
                    lov = jnp.where(m2, tv0_, rv2)
                    hik = jnp.where(m2, rk2, tk0)
                    hiv = jnp.where(m2, rv2, tv0_)
                    tk0, tv0_ = plsc.sort_key_val(lok, lov)
                    tk1, tv1_ = plsc.sort_key_val(hik, hiv)
                    return tk0, tk1, tv0_, tv1_
                return leaf

            carry = (inf16, inf16, zer16, zer16)
            carry = lax.fori_loop(0, (cna + 15) // 16,
                                  make_leaf(cia_v, zer16 + cna), carry)
            carry = lax.fori_loop(0, (cnb + 15) // 16,
                                  make_leaf(cib_v, zer16 + cnb), carry)
            _, _, acc0, acc1 = carry

            cp = (row0 - bi * n + r) * 3
            for h, acc in ((0, acc0), (1, acc1)):
                rl = (r * k + 16 * h) * 3 + iota16 * 3
                for d in range(3):
                    xs = plsc.load_gather(xyz_v, [acc * 3 + d])
                    cs = plsc.load_gather(xyz_v, [zer16 + (cp + d)])
                    plsc.store_scatter(nst_v, [rl + d], xs - cs)
                irs[slot][pl.ds(16 * h, 16)] = acc + boff

            @pl.when(r >= 2)
            def _wfree():
                pltpu.make_async_copy(
                    fbs[slot], feat_out.at[pl.ds(base + (r - 2) * k, k)],
                    wsems[slot]).wait()

            pltpu.async_copy(feat_hbm.at[irs[slot]], fbs[slot], fsems[slot])

            @pl.when(r >= 1)
            def _drain():
                other = 1 - slot
                pltpu.make_async_copy(feat_hbm.at[irs[other]], fbs[other],
                                      fsems[other]).wait()
                pltpu.async_copy(fbs[other],
                                 feat_out.at[pl.ds(base + (r - 1) * k, k)],
                                 wsems[other])

            @pl.when(r + 2 < pw)
            def _prefetch():
                pltpu.async_copy(dist_hbm.at[row0 + r + 2], dr, dsems[slot])

        @pl.loop(0, pw, step=2)
        def _(r0):
            for slot in range(2):
                process_point(r0 + slot, slot)

        lslot = (pw - 1) % 2
        pltpu.make_async_copy(
            fbs[1 - lslot], feat_out.at[pl.ds(base + (pw - 2) * k, k)],
            wsems[1 - lslot]).wait()
        pltpu.make_async_copy(feat_hbm.at[irs[lslot]], fbs[lslot],
                              fsems[lslot]).wait()
        pltpu.sync_copy(fbs[lslot],
                        feat_out.at[pl.ds(base + (pw - 1) * k, k)])
        pltpu.sync_copy(nst_v, neigh_out.at[pl.ds(base * 3, rw * 3)])

    return body(dist2, tau1, feat_flat, xyz2)


def kernel(xyz, feat):
    b, n, _ = xyz.shape
    c = feat.shape[-1]
    k = GROUP_SIZE
    dist, tau = _dist_tc(xyz)
    nfeat, neigh = _sc_select_gather(
        dist.reshape(b * n, n), tau.reshape(b * n), feat.reshape(b * n, c),
        xyz.reshape(b, n * 3), b=b, n=n, k=k, c=c)
    return (neigh.reshape(b, n, k, 3), nfeat.reshape(b, n, k, c))


# speedup vs baseline: 2.4940x; 1.0081x over previous
import functools

import jax
import jax.numpy as jnp
from jax import lax
from jax.experimental import pallas as pl
from jax.experimental.pallas import tpu as pltpu
from jax.experimental.pallas import tpu_sc as plsc

GROUP_SIZE = 32
BQ = 256

NC = 2
NS = 16
NW = NC * NS

BIGI = 1 << 30
FINF = float("inf")


def _dist_kernel(xyzq_ref, xyz_ref, dist_ref, tau_ref, *, n):
    q = xyzq_ref[0]
    ka = xyz_ref[0]
    sqq = jnp.sum(q * q, axis=1)
    sqk = jnp.sum(ka * ka, axis=1)
    inner = lax.dot_general(q, ka, (((1,), (1,)), ((), ())),
                            preferred_element_type=jnp.float32)
    dist = (sqq[:, None] + sqk[None, :]) - 2.0 * inner
    dist_ref[...] = dist
    nch = n // 128
    tau = jnp.min(lax.slice_in_dim(dist, 0, 128, axis=1), axis=1)
    for ci in range(1, nch):
        m = jnp.min(lax.slice_in_dim(dist, ci * 128, (ci + 1) * 128, axis=1),
                    axis=1)
        tau = jnp.maximum(tau, m)
    tau_ref[0, 0, :] = tau


def _dist_tc(xyz):
    b, n, _ = xyz.shape
    grid = (b, n // BQ)
    nq = n // BQ
    return pl.pallas_call(
        functools.partial(_dist_kernel, n=n),
        grid=grid,
        in_specs=[
            pl.BlockSpec((1, BQ, 3), lambda bi, qi: (bi, qi, 0)),
            pl.BlockSpec((1, n, 3), lambda bi, qi: (bi, 0, 0)),
        ],
        out_specs=(
            pl.BlockSpec((BQ, n), lambda bi, qi: (bi * nq + qi, 0)),
            pl.BlockSpec((1, 1, BQ), lambda bi, qi: (bi * nq + qi, 0, 0)),
        ),
        out_shape=(
            jax.ShapeDtypeStruct((b * n, n), jnp.float32),
            jax.ShapeDtypeStruct((b * nq, 1, BQ), jnp.float32),
        ),
    )(xyz, xyz)


def _sc_select_gather(dist2, tau1, feat_flat, xyz2, *, b, n, k, c):
    rtot = b * n * k
    rw = rtot // NW
    pw = rw // k
    nvreg = n // 16
    cap = n + 32

    mesh = plsc.VectorSubcoreMesh(core_axis_name="c", subcore_axis_name="s",
                                  num_cores=NC, num_subcores=NS)

    @functools.partial(
        pl.kernel, mesh=mesh,
        compiler_params=pltpu.CompilerParams(needs_layout_passes=False),
        out_type=(
            jax.ShapeDtypeStruct((rtot, c), jnp.float32),
            jax.ShapeDtypeStruct((rtot * 3,), jnp.float32),
        ),
        scratch_types=[
            pltpu.VMEM((n,), jnp.float32),
            pltpu.VMEM((n,), jnp.float32),
            pltpu.VMEM((cap,), jnp.int32),
            pltpu.VMEM((cap,), jnp.int32),
            pltpu.VMEM((pw + 16,), jnp.float32),
            pltpu.VMEM((n * 3,), jnp.float32),
            pltpu.VMEM((rw * 3,), jnp.float32),
            pltpu.VMEM((k,), jnp.int32),
            pltpu.VMEM((k,), jnp.int32),
            pltpu.VMEM((k, c), jnp.float32),
            pltpu.VMEM((k, c), jnp.float32),
            pltpu.SemaphoreType.DMA,
            pltpu.SemaphoreType.DMA,
            pltpu.SemaphoreType.DMA,
            pltpu.SemaphoreType.DMA,
            pltpu.SemaphoreType.DMA,
            pltpu.SemaphoreType.DMA,
        ],
    )
    def body(dist_hbm, tau_hbm, feat_hbm, xyz_hbm, feat_out, neigh_out,
             dr0, dr1, cia_v, cib_v, tau_v, xyz_v, nst_v,
             ir0, ir1, fb0, fb1, ds0, ds1, fs0, fs1, ws0, ws1):
        wid = lax.axis_index("s") * NC + lax.axis_index("c")
        bi = wid // (n // pw)
        row0 = wid * pw
        base = wid * rw
        boff = bi * n

        iota16 = lax.broadcasted_iota(jnp.int32, (16,), 0)
        zer16 = jnp.zeros((16,), jnp.int32)
        zer16f = jnp.zeros((16,), jnp.float32)
        inf16 = jnp.full((16,), FINF, jnp.float32)

        drs = (dr0, dr1)
        dsems = (ds0, ds1)
        irs = (ir0, ir1)
        fbs = (fb0, fb1)
        fsems = (fs0, fs1)
        wsems = (ws0, ws1)

        pltpu.sync_copy(tau_hbm.at[pl.ds(row0, pw)], tau_v.at[pl.ds(0, pw)])
        pltpu.sync_copy(xyz_hbm.at[bi], xyz_v)
        pltpu.async_copy(dist_hbm.at[row0], dr0, ds0)
        pltpu.async_copy(dist_hbm.at[row0 + 1], dr1, ds1)

        def process_point(r, slot):
            dr = drs[slot]
            pltpu.make_async_copy(dist_hbm.at[row0 + r], dr, dsems[slot]).wait()

            tv = tau_v[pl.ds(r, 16)]
            tau16 = zer16f + tv[0]

            half = n // 2

            def comp_body(i, carry):
                cna, cnb, civ = carry
                va = dr[pl.ds(i * 16, 16)]
                vb = dr[pl.ds(half + i * 16, 16)]
                mska = va <= tau16
                mskb = vb <= tau16
                plsc.store_compressed(cia_v.at[pl.ds(cna, 16)], civ,
                                      mask=mska)
                plsc.store_compressed(cib_v.at[pl.ds(cnb, 16)], civ + half,
                                      mask=mskb)
                pca = plsc.all_reduce_population_count(mska)
                pcb = plsc.all_reduce_population_count(mskb)
                return cna + pca[0], cnb + pcb[0], civ + 16

            cna, cnb, _ = lax.fori_loop(
                0, half // 16, comp_body,
                (jnp.int32(0), jnp.int32(0), iota16), unroll=8)
            cia_v[pl.ds(cna, 16)] = zer16
            cib_v[pl.ds(cnb, 16)] = zer16

            def make_leaf(buf, cnt16):
                def leaf(g, carry):
                    tk0, tk1, tv0_, tv1_ = carry
                    civ = buf[pl.ds(g * 16, 16)]
                    kv = plsc.load_gather(dr, [civ])
                    valid = (g * 16 + iota16) < cnt16
                    kv = jnp.where(valid, kv, inf16)
                    ks, vs = plsc.sort_key_val(kv, civ)
                    rk = jnp.flip(ks)
                    rv = jnp.flip(vs)
                    m = tk1 <= rk
                    ck = jnp.where(m, tk1, rk)
                    cw = jnp.where(m, tv1_, rv)
                    ck, cw = plsc.sort_key_val(ck, cw)
                    rk2 = jnp.flip(ck)
                    rv2 = jnp.flip(cw)
                    m2 = tk0 <= rk2
                    lok = jnp.where(m2, tk0, rk2)
                    lov = jnp.where(m2, tv0_, rv2)
                    hik = jnp.where(m2, rk2, tk0)
                    hiv = jnp.where(m2, rv2, tv0_)
                    tk0, tv0_ = plsc.sort_key_val(lok, lov)
                    tk1, tv1_ = plsc.sort_key_val(hik, hiv)
                    return tk0, tk1, tv0_, tv1_
                return leaf

            carry = (inf16, inf16, zer16, zer16)
            carry = lax.fori_loop(0, (cna + 15) // 16,
                                  make_leaf(cia_v, zer16 + cna), carry)
            carry = lax.fori_loop(0, (cnb + 15) // 16,
                                  make_leaf(cib_v, zer16 + cnb), carry)
            _, _, acc0, acc1 = carry

            cp = (row0 - bi * n + r) * 3
            cs3 = [plsc.load_gather(xyz_v, [zer16 + (cp + d)])
                   for d in range(3)]
            for h, acc in ((0, acc0), (1, acc1)):
                rl = (r * k + 16 * h) * 3 + iota16 * 3
                for d in range(3):
                    xs = plsc.load_gather(xyz_v, [acc * 3 + d])
                    plsc.store_scatter(nst_v, [rl + d], xs - cs3[d])
                irs[slot][pl.ds(16 * h, 16)] = acc + boff

            @pl.when(r >= 2)
            def _wfree():
                pltpu.make_async_copy(
                    fbs[slot], feat_out.at[pl.ds(base + (r - 2) * k, k)],
                    wsems[slot]).wait()

            pltpu.async_copy(feat_hbm.at[irs[slot]], fbs[slot], fsems[slot])

            @pl.when(r >= 1)
            def _drain():
                other = 1 - slot
                pltpu.make_async_copy(feat_hbm.at[irs[other]], fbs[other],
                                      fsems[other]).wait()
                pltpu.async_copy(fbs[other],
                                 feat_out.at[pl.ds(base + (r - 1) * k, k)],
                                 wsems[other])

            @pl.when(r + 2 < pw)
            def _prefetch():
                pltpu.async_copy(dist_hbm.at[row0 + r + 2], dr, dsems[slot])

        @pl.loop(0, pw, step=2)
        def _(r0):
            for slot in range(2):
                process_point(r0 + slot, slot)

        lslot = (pw - 1) % 2
        pltpu.make_async_copy(
            fbs[1 - lslot], feat_out.at[pl.ds(base + (pw - 2) * k, k)],
            wsems[1 - lslot]).wait()
        pltpu.make_async_copy(feat_hbm.at[irs[lslot]], fbs[lslot],
                              fsems[lslot]).wait()
        pltpu.sync_copy(fbs[lslot],
                        feat_out.at[pl.ds(base + (pw - 1) * k, k)])
        pltpu.sync_copy(nst_v, neigh_out.at[pl.ds(base * 3, rw * 3)])

    return body(dist2, tau1, feat_flat, xyz2)


def kernel(xyz, feat):
    b, n, _ = xyz.shape
    c = feat.shape[-1]
    k = GROUP_SIZE
    dist, tau = _dist_tc(xyz)
    nfeat, neigh = _sc_select_gather(
        dist.reshape(b * n, n), tau.reshape(b * n), feat.reshape(b * n, c),
        xyz.reshape(b, n * 3), b=b, n=n, k=k, c=c)
    return (neigh.reshape(b, n, k, 3), nfeat.reshape(b, n, k, c))
